# baseline (speedup 1.0000x reference)
import functools

import jax
import jax.numpy as jnp
from jax import lax
from jax.experimental import pallas as pl
from jax.experimental.pallas import tpu as pltpu
from jax.experimental.pallas import tpu_sc as plsc

D = 128
_NC = 2
_NS = 16
_NW = _NC * _NS
_CHUNK = 128
_H = 2
_BN = 2048
_HMASK = -65536


def _sc_gather_phase(value_vocab, path_vocab, idx_flat, n, h, nh):
    per_col = nh // _NW
    nch = per_col // _CHUNK
    mesh = plsc.VectorSubcoreMesh(core_axis_name="c", subcore_axis_name="s",
                                  num_cores=_NC)
    nbuf = 3

    @functools.partial(
        pl.kernel,
        mesh=mesh,
        out_type=jax.ShapeDtypeStruct((3 * nh, D // 2), jnp.int32),
        scratch_types=(
            [pltpu.VMEM((_CHUNK,), jnp.int32) for _ in range(nbuf)]
            + [pltpu.VMEM((_CHUNK, D), jnp.float32) for _ in range(nbuf)]
            + [pltpu.VMEM((_CHUNK, D // 2), jnp.int32) for _ in range(nbuf)]
            + [pltpu.SemaphoreType.DMA for _ in range(2 * nbuf)]
        ),
    )
    def gather_kernel(vv, pv, idxh, out, *scratch):
        idx_bufs = scratch[0:nbuf]
        row_bufs = scratch[nbuf:2 * nbuf]
        pk_bufs = scratch[2 * nbuf:3 * nbuf]
        gsems = scratch[3 * nbuf:4 * nbuf]
        wsems = scratch[4 * nbuf:5 * nbuf]
        wid = lax.axis_index("s") * _NC + lax.axis_index("c")
        tables = (vv, pv, vv)
        jobs = []
        for c in range(3):
            src0 = c * n + h * nh + wid * per_col
            dst0 = c * nh + wid * per_col
            for j in range(nch):
                jobs.append((src0 + j * _CHUNK, dst0 + j * _CHUNK, tables[c]))
        njobs = len(jobs)
        gcp = [None] * nbuf
        wcp = [None] * nbuf

        def fire_gather(k):
            src, _, table = jobs[k]
            b = k % nbuf
            pltpu.sync_copy(idxh.at[pl.ds(src, _CHUNK)], idx_bufs[b])
            gcp[b] = pltpu.async_copy(table.at[idx_bufs[b]], row_bufs[b], gsems[b])

        def pack_rows(b):
            rows_f, rows_p = row_bufs[b], pk_bufs[b]

            @plsc.parallel_loop(0, _CHUNK, unroll=4)
            def body(r):
                for g in range(4):
                    lo = lax.bitcast_convert_type(rows_f[r, pl.ds(g * 32, 16)],
                                                  jnp.int32)
                    hi = lax.bitcast_convert_type(
                        rows_f[r, pl.ds(g * 32 + 16, 16)], jnp.int32)
                    lo16 = lax.shift_right_logical(lo + 0x8000, 16)
                    hi16 = lax.bitwise_and(hi + 0x8000, _HMASK)
                    rows_p[r, pl.ds(g * 16, 16)] = lax.bitwise_or(hi16, lo16)

        for k in range(min(nbuf, njobs)):
            fire_gather(k)
        for k in range(njobs):
            b = k % nbuf
            gcp[b].wait()
            pack_rows(b)
            if k >= nbuf:
                wcp[b].wait()
            wcp[b] = pltpu.async_copy(pk_bufs[b],
                                      out.at[pl.ds(jobs[k][1], _CHUNK)], wsems[b])
            if k + nbuf < njobs:
                fire_gather(k + nbuf)
        for k in range(max(0, njobs - nbuf), njobs):
            wcp[k % nbuf].wait()

    return gather_kernel(value_vocab, path_vocab, idx_flat)


def _attn_body(ctx0, ctx1, ctx2, w_ref, a_ref, wd_ref, b_ref,
               m_in, s_in, v_in, m_out, s_out, v_out, *rest, nblk, is_last):
    if is_last:
        out_ref, m_ref, s_ref, v_ref = rest
    else:
        m_ref, s_ref, v_ref = rest
    i = pl.program_id(0)

    @pl.when(i == 0)
    def _():
        m_ref[0, 0] = m_in[0, 0]
        s_ref[0, 0] = s_in[0, 0]
        v_ref[...] = v_in[...]

    cw = None
    for c, ref in enumerate((ctx0, ctx1, ctx2)):
        x = ref[0]
        a_f = pltpu.bitcast(lax.shift_left(x, 16), jnp.float32)
        b_f = pltpu.bitcast(lax.bitwise_and(x, _HMASK), jnp.float32)
        xb = jnp.concatenate([a_f, b_f], axis=1).astype(jnp.bfloat16)
        wc = w_ref[:, c * D:(c + 1) * D].astype(jnp.bfloat16)
        part = lax.dot_general(xb, wc, (((1,), (1,)), ((), ())),
                               preferred_element_type=jnp.float32)
        cw = part if cw is None else cw + part
    combined = jnp.tanh(cw)
    z = lax.dot_general(combined, a_ref[...], (((1,), (1,)), ((), ())),
                        preferred_element_type=jnp.float32)
    m_old = m_ref[0, 0]
    m_new = jnp.maximum(m_old, jnp.max(z))
    corr = jnp.exp(m_old - m_new)
    p = jnp.exp(z - m_new)
    s_ref[0, 0] = s_ref[0, 0] * corr + jnp.sum(p)
    pv = lax.dot_general(p, combined, (((0,), (0,)), ((), ())),
                         preferred_element_type=jnp.float32)
    v_ref[...] = v_ref[...] * corr + pv
    m_ref[0, 0] = m_new

    @pl.when(i == nblk - 1)
    def _():
        m_out[0, 0] = m_ref[0, 0]
        s_out[0, 0] = s_ref[0, 0]
        v_out[...] = v_ref[...]
        if is_last:
            code = v_ref[...] / s_ref[0, 0]
            y = lax.dot_general(code, wd_ref[...], (((1,), (0,)), ((), ())),
                                preferred_element_type=jnp.float32) + b_ref[...]
            out_ref[...] = 1.0 / (1.0 + jnp.exp(-y))


def _tc_attn_phase(ctx3, W_perm, a_row, W_dense, b_row, state, nh, is_last):
    nblk = nh // _BN
    small = lambda i: (0, 0)
    out_shapes = [jax.ShapeDtypeStruct((1, 1), jnp.float32),
                  jax.ShapeDtypeStruct((1, 1), jnp.float32),
                  jax.ShapeDtypeStruct((1, D), jnp.float32)]
    smem_spec = pl.BlockSpec(memory_space=pltpu.SMEM)
    out_specs = [smem_spec,
                 smem_spec,
                 pl.BlockSpec((1, D), small)]
    if is_last:
        out_shapes.append(jax.ShapeDtypeStruct((1, D), jnp.float32))
        out_specs.append(pl.BlockSpec((1, D), small))
    return pl.pallas_call(
        functools.partial(_attn_body, nblk=nblk, is_last=is_last),
        grid=(nblk,),
        in_specs=[
            pl.BlockSpec((1, _BN, D // 2), lambda i: (0, i, 0)),
            pl.BlockSpec((1, _BN, D // 2), lambda i: (1, i, 0)),
            pl.BlockSpec((1, _BN, D // 2), lambda i: (2, i, 0)),
            pl.BlockSpec((D, 3 * D), small),
            pl.BlockSpec((1, D), small),
            pl.BlockSpec((D, D), small),
            pl.BlockSpec((1, D), small),
            smem_spec,
            smem_spec,
            pl.BlockSpec((1, D), small),
        ],
        out_specs=out_specs,
        out_shape=out_shapes,
        scratch_shapes=[
            pltpu.SMEM((1, 1), jnp.float32),
            pltpu.SMEM((1, 1), jnp.float32),
            pltpu.VMEM((1, D), jnp.float32),
        ],
    )(ctx3, ctx3, ctx3, W_perm, a_row, W_dense, b_row, *state)


def kernel(inputs, value_vocab, path_vocab, W, attention_vector, W_dense, b_dense):
    n = inputs.shape[0]
    nh = n // _H
    idx_flat = inputs.astype(jnp.int32).T.reshape(3 * n)
    W_perm = (W.reshape(D, 3, 4, 2, 16).transpose(0, 1, 3, 2, 4)
              .reshape(D, 3 * D))
    a_row = attention_vector.reshape(1, D)
    b_row = b_dense.reshape(1, D)
    state = (jnp.full((1, 1), -1e30, jnp.float32),
             jnp.zeros((1, 1), jnp.float32),
             jnp.zeros((1, D), jnp.float32))
    ctxs = [_sc_gather_phase(value_vocab, path_vocab, idx_flat, n, h, nh)
            for h in range(_H)]
    out = None
    for h in range(_H):
        ctx3 = ctxs[h].reshape(3, nh, D // 2)
        is_last = h == _H - 1
        res = _tc_attn_phase(ctx3, W_perm, a_row, W_dense, b_row, state, nh,
                             is_last)
        if is_last:
            out = res[3]
        state = res[:3]
    return out


# speedup vs baseline: 1.1040x; 1.1040x over previous
import functools

import jax
import jax.numpy as jnp
from jax import lax
from jax.experimental import pallas as pl
from jax.experimental.pallas import tpu as pltpu
from jax.experimental.pallas import tpu_sc as plsc

D = 128
_NC = 2
_NS = 16
_NW = _NC * _NS
_CHUNK = 128
_BN = 2048


def _sc_gather(value_vocab, path_vocab, idx_flat, n):
    per_col = n // _NW
    nch = per_col // _CHUNK
    mesh = plsc.VectorSubcoreMesh(core_axis_name="c", subcore_axis_name="s",
                                  num_cores=_NC)
    nbuf = 4

    @functools.partial(
        pl.kernel,
        mesh=mesh,
        out_type=jax.ShapeDtypeStruct((3 * n, D), jnp.float32),
        scratch_types=(
            [pltpu.VMEM((_CHUNK,), jnp.int32) for _ in range(nbuf)]
            + [pltpu.VMEM((_CHUNK, D), jnp.float32) for _ in range(nbuf)]
            + [pltpu.SemaphoreType.DMA for _ in range(2 * nbuf)]
        ),
    )
    def gather_kernel(vv, pv, idxh, out, *scratch):
        idx_bufs = scratch[0:nbuf]
        row_bufs = scratch[nbuf:2 * nbuf]
        gsems = scratch[2 * nbuf:3 * nbuf]
        wsems = scratch[3 * nbuf:4 * nbuf]
        wid = lax.axis_index("s") * _NC + lax.axis_index("c")
        tables = (vv, pv, vv)
        jobs = []
        for c in range(3):
            base0 = c * n + wid * per_col
            for j in range(nch):
                jobs.append((base0 + j * _CHUNK, tables[c]))
        njobs = len(jobs)
        gcp = [None] * nbuf
        wcp = [None] * nbuf

        def fire_gather(k):
            base, table = jobs[k]
            b = k % nbuf
            pltpu.sync_copy(idxh.at[pl.ds(base, _CHUNK)], idx_bufs[b])
            gcp[b] = pltpu.async_copy(table.at[idx_bufs[b]], row_bufs[b], gsems[b])

        for k in range(min(nbuf, njobs)):
            fire_gather(k)
        for k in range(njobs):
            b = k % nbuf
            gcp[b].wait()
            wcp[b] = pltpu.async_copy(row_bufs[b],
                                      out.at[pl.ds(jobs[k][0], _CHUNK)], wsems[b])
            if k + nbuf < njobs:
                wcp[b].wait()
                fire_gather(k + nbuf)
        for k in range(max(0, njobs - nbuf), njobs):
            wcp[k % nbuf].wait()

    return gather_kernel(value_vocab, path_vocab, idx_flat)


def _attn_body(ctx0, ctx1, ctx2, w_ref, a_ref, wd_ref, b_ref, out_ref,
               m_ref, s_ref, v_ref, *, nblk):
    i = pl.program_id(0)

    @pl.when(i == 0)
    def _():
        m_ref[0, 0] = -1e30
        s_ref[0, 0] = 0.0
        v_ref[...] = jnp.zeros_like(v_ref)

    cw = None
    for c, ref in enumerate((ctx0, ctx1, ctx2)):
        wc = w_ref[:, c * D:(c + 1) * D]
        part = lax.dot_general(ref[0], wc, (((1,), (1,)), ((), ())),
                               preferred_element_type=jnp.float32)
        cw = part if cw is None else cw + part
    combined = jnp.tanh(cw)
    z = lax.dot_general(combined, a_ref[...], (((1,), (1,)), ((), ())),
                        preferred_element_type=jnp.float32)
    m_old = m_ref[0, 0]
    m_new = jnp.maximum(m_old, jnp.max(z))
    corr = jnp.exp(m_old - m_new)
    p = jnp.exp(z - m_new)
    s_ref[0, 0] = s_ref[0, 0] * corr + jnp.sum(p)
    pv = lax.dot_general(p, combined, (((0,), (0,)), ((), ())),
                         preferred_element_type=jnp.float32)
    v_ref[...] = v_ref[...] * corr + pv
    m_ref[0, 0] = m_new

    @pl.when(i == nblk - 1)
    def _():
        code = v_ref[...] / s_ref[0, 0]
        y = lax.dot_general(code, wd_ref[...], (((1,), (0,)), ((), ())),
                            preferred_element_type=jnp.float32) + b_ref[...]
        out_ref[...] = 1.0 / (1.0 + jnp.exp(-y))


def _tc_attn(ctx3, W, a_row, W_dense, b_row, n):
    nblk = n // _BN
    small = lambda i: (0, 0)
    return pl.pallas_call(
        functools.partial(_attn_body, nblk=nblk),
        grid=(nblk,),
        in_specs=[
            pl.BlockSpec((1, _BN, D), lambda i: (0, i, 0)),
            pl.BlockSpec((1, _BN, D), lambda i: (1, i, 0)),
            pl.BlockSpec((1, _BN, D), lambda i: (2, i, 0)),
            pl.BlockSpec((D, 3 * D), small),
            pl.BlockSpec((1, D), small),
            pl.BlockSpec((D, D), small),
            pl.BlockSpec((1, D), small),
        ],
        out_specs=pl.BlockSpec((1, D), small),
        out_shape=jax.ShapeDtypeStruct((1, D), jnp.float32),
        scratch_shapes=[
            pltpu.SMEM((1, 1), jnp.float32),
            pltpu.SMEM((1, 1), jnp.float32),
            pltpu.VMEM((1, D), jnp.float32),
        ],
    )(ctx3, ctx3, ctx3, W, a_row, W_dense, b_row)


def kernel(inputs, value_vocab, path_vocab, W, attention_vector, W_dense, b_dense):
    n = inputs.shape[0]
    idx_flat = inputs.astype(jnp.int32).T.reshape(3 * n)
    ctx = _sc_gather(value_vocab, path_vocab, idx_flat, n)
    ctx3 = ctx.reshape(3, n, D)
    return _tc_attn(ctx3, W, attention_vector.reshape(1, D), W_dense,
                    b_dense.reshape(1, D), n)


# speedup vs baseline: 1.1540x; 1.0453x over previous
import functools

import jax
import jax.numpy as jnp
from jax import lax
from jax.experimental import pallas as pl
from jax.experimental.pallas import tpu as pltpu
from jax.experimental.pallas import tpu_sc as plsc

D = 128
_NC = 2
_NS = 16
_NW = _NC * _NS
_CHUNK = 128
_BN = 4096


def _sc_gather(value_vocab, path_vocab, idx_flat, n):
    per_col = n // _NW
    nch = per_col // _CHUNK
    mesh = plsc.VectorSubcoreMesh(core_axis_name="c", subcore_axis_name="s",
                                  num_cores=_NC)
    nbuf = 4

    @functools.partial(
        pl.kernel,
        mesh=mesh,
        out_type=jax.ShapeDtypeStruct((3 * n, D), jnp.float32),
        scratch_types=(
            [pltpu.VMEM((_CHUNK,), jnp.int32) for _ in range(nbuf)]
            + [pltpu.VMEM((_CHUNK, D), jnp.float32) for _ in range(nbuf)]
            + [pltpu.SemaphoreType.DMA for _ in range(2 * nbuf)]
        ),
    )
    def gather_kernel(vv, pv, idxh, out, *scratch):
        idx_bufs = scratch[0:nbuf]
        row_bufs = scratch[nbuf:2 * nbuf]
        gsems = scratch[2 * nbuf:3 * nbuf]
        wsems = scratch[3 * nbuf:4 * nbuf]
        wid = lax.axis_index("s") * _NC + lax.axis_index("c")
        tables = (vv, pv, vv)
        jobs = []
        for c in range(3):
            base0 = c * n + wid * per_col
            for j in range(nch):
                jobs.append((base0 + j * _CHUNK, tables[c]))
        njobs = len(jobs)
        gcp = [None] * nbuf
        wcp = [None] * nbuf

        def fire_gather(k):
            base, table = jobs[k]
            b = k % nbuf
            pltpu.sync_copy(idxh.at[pl.ds(base, _CHUNK)], idx_bufs[b])
            gcp[b] = pltpu.async_copy(table.at[idx_bufs[b]], row_bufs[b], gsems[b])

        for k in range(min(nbuf, njobs)):
            fire_gather(k)
        for k in range(njobs):
            b = k % nbuf
            gcp[b].wait()
            wcp[b] = pltpu.async_copy(row_bufs[b],
                                      out.at[pl.ds(jobs[k][0], _CHUNK)], wsems[b])
            if k + nbuf < njobs:
                wcp[b].wait()
                fire_gather(k + nbuf)
        for k in range(max(0, njobs - nbuf), njobs):
            wcp[k % nbuf].wait()

    return gather_kernel(value_vocab, path_vocab, idx_flat)


def _attn_body(ctx0, ctx1, ctx2, w_ref, a_ref, wd_ref, b_ref, out_ref,
               m_ref, s_ref, v_ref, *, nblk):
    i = pl.program_id(0)

    @pl.when(i == 0)
    def _():
        m_ref[0, 0] = -1e30
        s_ref[0, 0] = 0.0
        v_ref[...] = jnp.zeros_like(v_ref)

    cw = None
    for c, ref in enumerate((ctx0, ctx1, ctx2)):
        wc = w_ref[:, c * D:(c + 1) * D]
        part = lax.dot_general(ref[0], wc, (((1,), (1,)), ((), ())),
                               preferred_element_type=jnp.float32)
        cw = part if cw is None else cw + part
    combined = jnp.tanh(cw)
    z = lax.dot_general(combined, a_ref[...], (((1,), (1,)), ((), ())),
                        preferred_element_type=jnp.float32)
    m_old = m_ref[0, 0]
    m_new = jnp.maximum(m_old, jnp.max(z))
    corr = jnp.exp(m_old - m_new)
    p = jnp.exp(z - m_new)
    s_ref[0, 0] = s_ref[0, 0] * corr + jnp.sum(p)
    pv = lax.dot_general(p, combined, (((0,), (0,)), ((), ())),
                         preferred_element_type=jnp.float32)
    v_ref[...] = v_ref[...] * corr + pv
    m_ref[0, 0] = m_new

    @pl.when(i == nblk - 1)
    def _():
        code = v_ref[...] / s_ref[0, 0]
        y = lax.dot_general(code, wd_ref[...], (((1,), (0,)), ((), ())),
                            preferred_element_type=jnp.float32) + b_ref[...]
        out_ref[...] = 1.0 / (1.0 + jnp.exp(-y))


def _tc_attn(ctx3, W, a_row, W_dense, b_row, n):
    nblk = n // _BN
    small = lambda i: (0, 0)
    return pl.pallas_call(
        functools.partial(_attn_body, nblk=nblk),
        grid=(nblk,),
        in_specs=[
            pl.BlockSpec((1, _BN, D), lambda i: (0, i, 0)),
            pl.BlockSpec((1, _BN, D), lambda i: (1, i, 0)),
            pl.BlockSpec((1, _BN, D), lambda i: (2, i, 0)),
            pl.BlockSpec((D, 3 * D), small),
            pl.BlockSpec((1, D), small),
            pl.BlockSpec((D, D), small),
            pl.BlockSpec((1, D), small),
        ],
        out_specs=pl.BlockSpec((1, D), small),
        out_shape=jax.ShapeDtypeStruct((1, D), jnp.float32),
        scratch_shapes=[
            pltpu.SMEM((1, 1), jnp.float32),
            pltpu.SMEM((1, 1), jnp.float32),
            pltpu.VMEM((1, D), jnp.float32),
        ],
    )(ctx3, ctx3, ctx3, W, a_row, W_dense, b_row)


def kernel(inputs, value_vocab, path_vocab, W, attention_vector, W_dense, b_dense):
    n = inputs.shape[0]
    idx_flat = inputs.astype(jnp.int32).T.reshape(3 * n)
    ctx = _sc_gather(value_vocab, path_vocab, idx_flat, n)
    ctx3 = ctx.reshape(3, n, D)
    return _tc_attn(ctx3, W, attention_vector.reshape(1, D), W_dense,
                    b_dense.reshape(1, D), n)
